# baseline (speedup 1.0000x reference)
import jax
import jax.numpy as jnp
from jax import lax
from jax.experimental import pallas as pl
from jax.experimental.pallas import tpu as pltpu
from jax.experimental.pallas import tpu_sc as plsc

BATCH = 16384
EMBED = 32
NUM_CORES = 2
NUM_SUBCORES = 16
NUM_WORKERS = NUM_CORES * NUM_SUBCORES
B_PER_W = BATCH // NUM_WORKERS
IDX_CHUNK = 128
N_CHUNKS = B_PER_W // IDX_CHUNK
LANES = 16
N_GROUPS = B_PER_W // LANES


def _sc_body(uid_hbm, iid_hbm, user_hbm, item_hbm, out_hbm,
             uidx, iidx, urows, irows, outv, sem):
    wid = lax.axis_index("s") * NUM_CORES + lax.axis_index("c")
    base = wid * B_PER_W

    pltpu.sync_copy(uid_hbm.at[wid], uidx)
    pltpu.sync_copy(iid_hbm.at[wid], iidx)

    copies = []
    for j in range(N_CHUNKS):
        dst = pl.ds(j * IDX_CHUNK, IDX_CHUNK)
        copies.append(pltpu.async_copy(user_hbm.at[uidx.at[j]], urows.at[dst], sem))
        copies.append(pltpu.async_copy(item_hbm.at[iidx.at[j]], irows.at[dst], sem))
    for c in copies:
        c.wait()

    def group(g, carry):
        rows = g * LANES + lax.iota(jnp.int32, LANES)
        acc = jnp.zeros((LANES,), jnp.float32)
        for d in range(EMBED):
            col = jnp.full((LANES,), d, jnp.int32)
            ug = plsc.load_gather(urows, [rows, col])
            vg = plsc.load_gather(irows, [rows, col])
            acc = acc + ug * vg
        outv[pl.ds(g * LANES, LANES)] = acc
        return carry

    lax.fori_loop(0, N_GROUPS, group, 0)

    pltpu.sync_copy(outv, out_hbm.at[pl.ds(base, B_PER_W)])


@jax.jit
def _sc_call(uid3, iid3, user_memory, item_memory):
    mesh = plsc.VectorSubcoreMesh(core_axis_name="c", subcore_axis_name="s")
    return pl.kernel(
        _sc_body,
        out_type=jax.ShapeDtypeStruct((BATCH,), jnp.float32),
        mesh=mesh,
        scratch_types=[
            pltpu.VMEM((N_CHUNKS, IDX_CHUNK), jnp.int32),
            pltpu.VMEM((N_CHUNKS, IDX_CHUNK), jnp.int32),
            pltpu.VMEM((B_PER_W, EMBED), jnp.float32),
            pltpu.VMEM((B_PER_W, EMBED), jnp.float32),
            pltpu.VMEM((B_PER_W,), jnp.float32),
            pltpu.SemaphoreType.DMA,
        ],
        compiler_params=pltpu.CompilerParams(
            needs_layout_passes=False, use_tc_tiling_on_sc=False
        ),
    )(uid3, iid3, user_memory, item_memory)


def kernel(batch, user_memory, item_memory):
    uid3 = batch[:, 0].reshape(NUM_WORKERS, N_CHUNKS, IDX_CHUNK)
    iid3 = batch[:, 1].reshape(NUM_WORKERS, N_CHUNKS, IDX_CHUNK)
    return _sc_call(uid3, iid3, user_memory, item_memory)
